# baseline (speedup 1.0000x reference)
import functools

import jax
import jax.numpy as jnp
from jax import lax
from jax.experimental import pallas as pl
from jax.experimental.pallas import tpu as pltpu
from jax.experimental.pallas import tpu_sc as plsc

N = 327680
V = 100000
D = 128
H = D // 2

NC = 2
NS = 16
NW = NC * NS
B_PER_W = N // NW
CHUNK = 256
NCHUNK = B_PER_W // CHUNK

ROWS_TC = 4096


def _sc_gather_two(act_table, res_table, act_idx, res_idx):
    mesh = plsc.VectorSubcoreMesh(core_axis_name="c", subcore_axis_name="s")
    out_t = jax.ShapeDtypeStruct((N, D), jnp.float32)

    @functools.partial(
        pl.kernel, mesh=mesh, out_type=out_t,
        compiler_params=pltpu.CompilerParams(use_tc_tiling_on_sc=False),
        scratch_types=(
            [pltpu.VMEM((B_PER_W,), jnp.int32)] * 2
            + [pltpu.VMEM((CHUNK, H), jnp.float32)] * 4
            + [pltpu.SemaphoreType.DMA] * 8
        ),
    )
    def k(act_tab, res_tab, ai_hbm, ri_hbm, cat_hbm,
          idx_av, idx_rv, ra0, ra1, rr0, rr1,
          sga0, sga1, sgr0, sgr1, swa0, swa1, swr0, swr1):
        wid = lax.axis_index("s") * NC + lax.axis_index("c")
        rows_a = (ra0, ra1)
        rows_r = (rr0, rr1)
        sem_ga = (sga0, sga1)
        sem_gr = (sgr0, sgr1)
        sem_wa = (swa0, swa1)
        sem_wr = (swr0, swr1)
        wbase = wid * B_PER_W

        pltpu.sync_copy(ai_hbm.at[pl.ds(wbase, B_PER_W)], idx_av)
        pltpu.sync_copy(ri_hbm.at[pl.ds(wbase, B_PER_W)], idx_rv)

        def g_copies(j, b):
            s = pl.ds(j * CHUNK, CHUNK)
            return (
                pltpu.make_async_copy(act_tab.at[idx_av.at[s]], rows_a[b],
                                      sem_ga[b]),
                pltpu.make_async_copy(res_tab.at[idx_rv.at[s]], rows_r[b],
                                      sem_gr[b]),
            )

        def w_copies(j, b):
            g = pl.ds(wbase + j * CHUNK, CHUNK)
            return (
                pltpu.make_async_copy(rows_a[b], cat_hbm.at[g, pl.ds(0, H)],
                                      sem_wa[b]),
                pltpu.make_async_copy(rows_r[b], cat_hbm.at[g, pl.ds(H, H)],
                                      sem_wr[b]),
            )

        def issue(cs):
            for c in cs:
                c.start()

        def wait(cs):
            for c in cs:
                c.wait()

        issue(g_copies(0, 0))

        @pl.loop(0, NCHUNK, step=2)
        def _(i):
            for b in (0, 1):
                j = i + b
                wait(g_copies(j, b))
                issue(w_copies(j, b))
                nx = j + 1

                @pl.when(nx < NCHUNK)
                def _():
                    @pl.when(nx >= 2)
                    def _():
                        wait(w_copies(nx - 2, 1 - b))

                    issue(g_copies(nx, 1 - b))

        wait(w_copies(NCHUNK - 2, 0))
        wait(w_copies(NCHUNK - 1, 1))

    return k(act_table, res_table, act_idx, res_idx)


def _tc_dense_body(cat_ref, nf_ref, w1_ref, b1_ref, oc_ref,
                   wc_ref, wn_ref, pb_ref, g2_ref, be2_ref, out_ref):
    dot = functools.partial(jnp.dot, preferred_element_type=jnp.float32)
    h = jnp.maximum(dot(nf_ref[...], w1_ref[...]) + b1_ref[...], 0.0)
    hc = h - dot(h, oc_ref[...])
    v1 = dot(hc * hc, oc_ref[...])
    z1 = hc * lax.rsqrt(v1 + 1e-5)
    fc = dot(cat_ref[...], wc_ref[...]) + dot(z1, wn_ref[...]) + pb_ref[...]
    v2 = dot(fc * fc, oc_ref[...])
    out_ref[...] = fc * lax.rsqrt(v2 + 1e-5) * g2_ref[...] + be2_ref[...]


def _tc_dense(cat_emb, nf, w1, b1, onesc, wc, wn, pb, g2, be2):
    grid = (N // ROWS_TC,)
    row_spec = lambda c: pl.BlockSpec((ROWS_TC, c), lambda i: (i, 0))
    full_spec = lambda a: pl.BlockSpec(a.shape, lambda i: (0, 0))
    return pl.pallas_call(
        _tc_dense_body,
        grid=grid,
        in_specs=[
            row_spec(D), row_spec(3),
            full_spec(w1), full_spec(b1), full_spec(onesc),
            full_spec(wc), full_spec(wn),
            full_spec(pb), full_spec(g2), full_spec(be2),
        ],
        out_specs=row_spec(D),
        out_shape=jax.ShapeDtypeStruct((N, D), jnp.float32),
    )(cat_emb, nf, w1, b1, onesc, wc, wn, pb, g2, be2)


def kernel(activities, resources, num_feats, act_table, res_table, num_w1,
           num_b1, ln1_g, ln1_b, proj_w, proj_b, ln2_g, ln2_b):
    ai = activities.astype(jnp.int32)
    ri = resources.astype(jnp.int32)
    cat_emb = _sc_gather_two(act_table, res_table, ai, ri)

    mm = jnp.eye(D, dtype=jnp.float32) - 1.0 / D
    onesc = jnp.full((D, 1), 1.0 / D, jnp.float32)

    wn_raw = proj_w[2 * H:]
    wc = proj_w[0:2 * H] @ mm
    wn = (ln1_g[:, None] * wn_raw) @ mm
    pb = ((ln1_b @ wn_raw + proj_b) @ mm)[None, :]
    return _tc_dense(cat_emb, num_feats, num_w1, num_b1[None, :], onesc,
                     wc, wn, pb, ln2_g.reshape(1, D), ln2_b.reshape(1, D))


# speedup vs baseline: 1.2617x; 1.2617x over previous
import functools

import jax
import jax.numpy as jnp
from jax import lax
from jax.experimental import pallas as pl
from jax.experimental.pallas import tpu as pltpu
from jax.experimental.pallas import tpu_sc as plsc

N = 327680
V = 100000
D = 128
H = D // 2

NC = 2
NS = 16
NW = NC * NS
B_PER_W = N // NW
CHUNK = 320
NCHUNK = B_PER_W // CHUNK

ROWS_TC = 8192


def _sc_gather_two(act_table, res_table, act_idx, res_idx):
    mesh = plsc.VectorSubcoreMesh(core_axis_name="c", subcore_axis_name="s")
    out_t = jax.ShapeDtypeStruct((N, D), jnp.float32)

    @functools.partial(
        pl.kernel, mesh=mesh, out_type=out_t,
        compiler_params=pltpu.CompilerParams(use_tc_tiling_on_sc=False),
        scratch_types=(
            [pltpu.VMEM((B_PER_W,), jnp.int32)] * 2
            + [pltpu.VMEM((CHUNK, H), jnp.float32)] * 4
            + [pltpu.SemaphoreType.DMA] * 8
        ),
    )
    def k(act_tab, res_tab, ai_hbm, ri_hbm, cat_hbm,
          idx_av, idx_rv, ra0, ra1, rr0, rr1,
          sga0, sga1, sgr0, sgr1, swa0, swa1, swr0, swr1):
        wid = lax.axis_index("s") * NC + lax.axis_index("c")
        rows_a = (ra0, ra1)
        rows_r = (rr0, rr1)
        sem_ga = (sga0, sga1)
        sem_gr = (sgr0, sgr1)
        sem_wa = (swa0, swa1)
        sem_wr = (swr0, swr1)
        wbase = wid * B_PER_W

        pltpu.sync_copy(ai_hbm.at[pl.ds(wbase, B_PER_W)], idx_av)
        pltpu.sync_copy(ri_hbm.at[pl.ds(wbase, B_PER_W)], idx_rv)

        def g_copies(j, b):
            s = pl.ds(j * CHUNK, CHUNK)
            return (
                pltpu.make_async_copy(act_tab.at[idx_av.at[s]], rows_a[b],
                                      sem_ga[b]),
                pltpu.make_async_copy(res_tab.at[idx_rv.at[s]], rows_r[b],
                                      sem_gr[b]),
            )

        def w_copies(j, b):
            g = pl.ds(wbase + j * CHUNK, CHUNK)
            return (
                pltpu.make_async_copy(rows_a[b], cat_hbm.at[g, pl.ds(0, H)],
                                      sem_wa[b]),
                pltpu.make_async_copy(rows_r[b], cat_hbm.at[g, pl.ds(H, H)],
                                      sem_wr[b]),
            )

        def issue(cs):
            for c in cs:
                c.start()

        def wait(cs):
            for c in cs:
                c.wait()

        issue(g_copies(0, 0))

        @pl.loop(0, NCHUNK, step=2)
        def _(i):
            for b in (0, 1):
                j = i + b
                wait(g_copies(j, b))
                issue(w_copies(j, b))
                nx = j + 1

                @pl.when(nx < NCHUNK)
                def _():
                    @pl.when(nx >= 2)
                    def _():
                        wait(w_copies(nx - 2, 1 - b))

                    issue(g_copies(nx, 1 - b))

        wait(w_copies(NCHUNK - 2, 0))
        wait(w_copies(NCHUNK - 1, 1))

    return k(act_table, res_table, act_idx, res_idx)


def _tc_dense_body(cat_ref, nft_ref, w1t_ref, or_ref, oc_ref,
                   wc_ref, wnt_ref, pb_ref, g2_ref, be2_ref, out_ref):
    dot = functools.partial(jnp.dot, preferred_element_type=jnp.float32)
    ht = jnp.maximum(dot(w1t_ref[...], nft_ref[...]), 0.0)
    hct = ht - dot(or_ref[...], ht)
    v1t = dot(or_ref[...], hct * hct)
    z1t = hct * lax.rsqrt(v1t + 1e-5)
    u = dot(wnt_ref[...], z1t).T
    fc = dot(cat_ref[...], wc_ref[...]) + u + pb_ref[...]
    v2 = dot(fc * fc, oc_ref[...])
    out_ref[...] = fc * lax.rsqrt(v2 + 1e-5) * g2_ref[...] + be2_ref[...]


def _tc_dense(cat_emb, nft, w1t, onesr, onesc, wc, wnt, pb, g2, be2):
    grid = (N // ROWS_TC,)
    row_spec = lambda c: pl.BlockSpec((ROWS_TC, c), lambda i: (i, 0))
    full_spec = lambda a: pl.BlockSpec(a.shape, lambda i: (0, 0))
    return pl.pallas_call(
        _tc_dense_body,
        grid=grid,
        in_specs=[
            row_spec(D), pl.BlockSpec((8, ROWS_TC), lambda i: (0, i)),
            full_spec(w1t), full_spec(onesr), full_spec(onesc),
            full_spec(wc), full_spec(wnt),
            full_spec(pb), full_spec(g2), full_spec(be2),
        ],
        out_specs=row_spec(D),
        out_shape=jax.ShapeDtypeStruct((N, D), jnp.float32),
    )(cat_emb, nft, w1t, onesr, onesc, wc, wnt, pb, g2, be2)


def kernel(activities, resources, num_feats, act_table, res_table, num_w1,
           num_b1, ln1_g, ln1_b, proj_w, proj_b, ln2_g, ln2_b):
    ai = activities.astype(jnp.int32)
    ri = resources.astype(jnp.int32)
    cat_emb = _sc_gather_two(act_table, res_table, ai, ri)

    nft = jnp.concatenate(
        [num_feats.T, jnp.ones((1, N), jnp.float32),
         jnp.zeros((4, N), jnp.float32)], axis=0)
    w1t = jnp.concatenate(
        [num_w1, num_b1[None, :], jnp.zeros((4, D), jnp.float32)], axis=0).T

    mm = jnp.eye(D, dtype=jnp.float32) - 1.0 / D
    onesr = jnp.full((1, D), 1.0 / D, jnp.float32)
    onesc = jnp.full((D, 1), 1.0 / D, jnp.float32)

    wn_raw = proj_w[2 * H:]
    wc = proj_w[0:2 * H] @ mm
    wnt = ((ln1_g[:, None] * wn_raw) @ mm).T
    pb = ((ln1_b @ wn_raw + proj_b) @ mm)[None, :]
    return _tc_dense(cat_emb, nft, w1t, onesr, onesc, wc, wnt, pb,
                     ln2_g.reshape(1, D), ln2_b.reshape(1, D))


# speedup vs baseline: 1.3140x; 1.0415x over previous
import functools

import jax
import jax.numpy as jnp
from jax import lax
from jax.experimental import pallas as pl
from jax.experimental.pallas import tpu as pltpu
from jax.experimental.pallas import tpu_sc as plsc

N = 327680
V = 100000
D = 128
H = D // 2

NC = 2
NS = 16
NW = NC * NS
SLABS = 2
NSL = N // SLABS
B_PER_W = NSL // NW
CHUNK = 320
NCHUNK = B_PER_W // CHUNK

ROWS_TC = 8192


def _sc_gather_two(act_table, res_table, act_idx, res_idx):
    mesh = plsc.VectorSubcoreMesh(core_axis_name="c", subcore_axis_name="s")
    out_t = jax.ShapeDtypeStruct((NSL, D), jnp.float32)

    @functools.partial(
        pl.kernel, mesh=mesh, out_type=out_t,
        compiler_params=pltpu.CompilerParams(use_tc_tiling_on_sc=False),
        scratch_types=(
            [pltpu.VMEM((B_PER_W,), jnp.int32)] * 2
            + [pltpu.VMEM((CHUNK, H), jnp.float32)] * 4
            + [pltpu.SemaphoreType.DMA] * 8
        ),
    )
    def k(act_tab, res_tab, ai_hbm, ri_hbm, cat_hbm,
          idx_av, idx_rv, ra0, ra1, rr0, rr1,
          sga0, sga1, sgr0, sgr1, swa0, swa1, swr0, swr1):
        wid = lax.axis_index("s") * NC + lax.axis_index("c")
        rows_a = (ra0, ra1)
        rows_r = (rr0, rr1)
        sem_ga = (sga0, sga1)
        sem_gr = (sgr0, sgr1)
        sem_wa = (swa0, swa1)
        sem_wr = (swr0, swr1)
        wbase = wid * B_PER_W

        pltpu.sync_copy(ai_hbm.at[pl.ds(wbase, B_PER_W)], idx_av)
        pltpu.sync_copy(ri_hbm.at[pl.ds(wbase, B_PER_W)], idx_rv)

        def g_copies(j, b):
            s = pl.ds(j * CHUNK, CHUNK)
            return (
                pltpu.make_async_copy(act_tab.at[idx_av.at[s]], rows_a[b],
                                      sem_ga[b]),
                pltpu.make_async_copy(res_tab.at[idx_rv.at[s]], rows_r[b],
                                      sem_gr[b]),
            )

        def w_copies(j, b):
            g = pl.ds(wbase + j * CHUNK, CHUNK)
            return (
                pltpu.make_async_copy(rows_a[b], cat_hbm.at[g, pl.ds(0, H)],
                                      sem_wa[b]),
                pltpu.make_async_copy(rows_r[b], cat_hbm.at[g, pl.ds(H, H)],
                                      sem_wr[b]),
            )

        def issue(cs):
            for c in cs:
                c.start()

        def wait(cs):
            for c in cs:
                c.wait()

        issue(g_copies(0, 0))

        @pl.loop(0, NCHUNK, step=2)
        def _(i):
            for b in (0, 1):
                j = i + b
                wait(g_copies(j, b))
                issue(w_copies(j, b))
                nx = j + 1

                @pl.when(nx < NCHUNK)
                def _():
                    @pl.when(nx >= 2)
                    def _():
                        wait(w_copies(nx - 2, 1 - b))

                    issue(g_copies(nx, 1 - b))

        wait(w_copies(NCHUNK - 2, 0))
        wait(w_copies(NCHUNK - 1, 1))

    return k(act_table, res_table, act_idx, res_idx)


def _tc_dense_body(cat_ref, nft_ref, w1t_ref, or_ref, oc_ref,
                   wc_ref, wnt_ref, pb_ref, g2_ref, be2_ref, out_ref):
    dot = functools.partial(jnp.dot, preferred_element_type=jnp.float32)
    ht = jnp.maximum(dot(w1t_ref[...], nft_ref[...]), 0.0)
    hct = ht - dot(or_ref[...], ht)
    v1t = dot(or_ref[...], hct * hct)
    z1t = hct * lax.rsqrt(v1t + 1e-5)
    u = dot(wnt_ref[...], z1t).T
    fc = dot(cat_ref[...], wc_ref[...]) + u + pb_ref[...]
    v2 = dot(fc * fc, oc_ref[...])
    out_ref[...] = fc * lax.rsqrt(v2 + 1e-5) * g2_ref[...] + be2_ref[...]


def _tc_dense(cat_emb, nft, w1t, onesr, onesc, wc, wnt, pb, g2, be2,
              slab, buf=None):
    tiles = NSL // ROWS_TC
    off = slab * tiles
    grid = (tiles,)
    row_spec = lambda c: pl.BlockSpec((ROWS_TC, c), lambda i: (i, 0))
    full_spec = lambda a: pl.BlockSpec(a.shape, lambda i: (0, 0))
    in_specs = [
        row_spec(D), pl.BlockSpec((8, ROWS_TC), lambda i: (0, i + off)),
        full_spec(w1t), full_spec(onesr), full_spec(onesc),
        full_spec(wc), full_spec(wnt),
        full_spec(pb), full_spec(g2), full_spec(be2),
    ]
    args = [cat_emb, nft, w1t, onesr, onesc, wc, wnt, pb, g2, be2]
    kwargs = {}
    body = _tc_dense_body
    if buf is not None:
        in_specs.append(pl.BlockSpec(memory_space=pl.ANY))
        args.append(buf)
        kwargs["input_output_aliases"] = {10: 0}
        body = lambda *refs: _tc_dense_body(*refs[:10], refs[-1])
    return pl.pallas_call(
        body,
        grid=grid,
        in_specs=in_specs,
        out_specs=pl.BlockSpec((ROWS_TC, D), lambda i: (i + off, 0)),
        out_shape=jax.ShapeDtypeStruct((N, D), jnp.float32),
        **kwargs,
    )(*args)


def kernel(activities, resources, num_feats, act_table, res_table, num_w1,
           num_b1, ln1_g, ln1_b, proj_w, proj_b, ln2_g, ln2_b):
    ai = activities.astype(jnp.int32)
    ri = resources.astype(jnp.int32)
    cats = [_sc_gather_two(act_table, res_table,
                           ai[s * NSL:(s + 1) * NSL], ri[s * NSL:(s + 1) * NSL])
            for s in range(SLABS)]

    nft = jnp.concatenate(
        [num_feats.T, jnp.ones((1, N), jnp.float32),
         jnp.zeros((4, N), jnp.float32)], axis=0)
    w1t = jnp.concatenate(
        [num_w1, num_b1[None, :], jnp.zeros((4, D), jnp.float32)], axis=0).T

    mm = jnp.eye(D, dtype=jnp.float32) - 1.0 / D
    onesr = jnp.full((1, D), 1.0 / D, jnp.float32)
    onesc = jnp.full((D, 1), 1.0 / D, jnp.float32)

    wn_raw = proj_w[2 * H:]
    wc = proj_w[0:2 * H] @ mm
    wnt = ((ln1_g[:, None] * wn_raw) @ mm).T
    pb = ((ln1_b @ wn_raw + proj_b) @ mm)[None, :]
    g2 = ln2_g.reshape(1, D)
    b2 = ln2_b.reshape(1, D)
    out = None
    for s in range(SLABS):
        out = _tc_dense(cats[s], nft, w1t, onesr, onesc, wc, wnt, pb,
                        g2, b2, slab=s, buf=out)
    return out


# speedup vs baseline: 1.3289x; 1.0113x over previous
import functools

import jax
import jax.numpy as jnp
from jax import lax
from jax.experimental import pallas as pl
from jax.experimental.pallas import tpu as pltpu
from jax.experimental.pallas import tpu_sc as plsc

N = 327680
V = 100000
D = 128
H = D // 2

NC = 2
NS = 16
NW = NC * NS
SLABS = 4
NSL = N // SLABS
B_PER_W = NSL // NW
CHUNK = 320
NCHUNK = B_PER_W // CHUNK

ROWS_TC = 8192


def _sc_gather_two(act_table, res_table, act_idx, res_idx):
    mesh = plsc.VectorSubcoreMesh(core_axis_name="c", subcore_axis_name="s")
    out_t = jax.ShapeDtypeStruct((NSL, D), jnp.float32)

    @functools.partial(
        pl.kernel, mesh=mesh, out_type=out_t,
        compiler_params=pltpu.CompilerParams(use_tc_tiling_on_sc=False),
        scratch_types=(
            [pltpu.VMEM((B_PER_W,), jnp.int32)] * 2
            + [pltpu.VMEM((CHUNK, H), jnp.float32)] * 4
            + [pltpu.SemaphoreType.DMA] * 8
        ),
    )
    def k(act_tab, res_tab, ai_hbm, ri_hbm, cat_hbm,
          idx_av, idx_rv, ra0, ra1, rr0, rr1,
          sga0, sga1, sgr0, sgr1, swa0, swa1, swr0, swr1):
        wid = lax.axis_index("s") * NC + lax.axis_index("c")
        rows_a = (ra0, ra1)
        rows_r = (rr0, rr1)
        sem_ga = (sga0, sga1)
        sem_gr = (sgr0, sgr1)
        sem_wa = (swa0, swa1)
        sem_wr = (swr0, swr1)
        wbase = wid * B_PER_W

        pltpu.sync_copy(ai_hbm.at[pl.ds(wbase, B_PER_W)], idx_av)
        pltpu.sync_copy(ri_hbm.at[pl.ds(wbase, B_PER_W)], idx_rv)

        def g_copies(j, b):
            s = pl.ds(j * CHUNK, CHUNK)
            return (
                pltpu.make_async_copy(act_tab.at[idx_av.at[s]], rows_a[b],
                                      sem_ga[b]),
                pltpu.make_async_copy(res_tab.at[idx_rv.at[s]], rows_r[b],
                                      sem_gr[b]),
            )

        def w_copies(j, b):
            g = pl.ds(wbase + j * CHUNK, CHUNK)
            return (
                pltpu.make_async_copy(rows_a[b], cat_hbm.at[g, pl.ds(0, H)],
                                      sem_wa[b]),
                pltpu.make_async_copy(rows_r[b], cat_hbm.at[g, pl.ds(H, H)],
                                      sem_wr[b]),
            )

        def issue(cs):
            for c in cs:
                c.start()

        def wait(cs):
            for c in cs:
                c.wait()

        issue(g_copies(0, 0))

        @pl.loop(0, NCHUNK, step=2)
        def _(i):
            for b in (0, 1):
                j = i + b
                wait(g_copies(j, b))
                issue(w_copies(j, b))
                nx = j + 1

                @pl.when(nx < NCHUNK)
                def _():
                    @pl.when(nx >= 2)
                    def _():
                        wait(w_copies(nx - 2, 1 - b))

                    issue(g_copies(nx, 1 - b))

        wait(w_copies(NCHUNK - 2, 0))
        wait(w_copies(NCHUNK - 1, 1))

    return k(act_table, res_table, act_idx, res_idx)


def _tc_dense_body(cat_ref, nft_ref, w1t_ref, or_ref, oc_ref,
                   wc_ref, wnt_ref, pb_ref, g2_ref, be2_ref, out_ref):
    dot = functools.partial(jnp.dot, preferred_element_type=jnp.float32)
    ht = jnp.maximum(dot(w1t_ref[...], nft_ref[...]), 0.0)
    hct = ht - dot(or_ref[...], ht)
    v1t = dot(or_ref[...], hct * hct)
    z1t = hct * lax.rsqrt(v1t + 1e-5)
    u = dot(wnt_ref[...], z1t).T
    fc = dot(cat_ref[...], wc_ref[...]) + u + pb_ref[...]
    v2 = dot(fc * fc, oc_ref[...])
    out_ref[...] = fc * lax.rsqrt(v2 + 1e-5) * g2_ref[...] + be2_ref[...]


def _tc_dense(cat_emb, nft, w1t, onesr, onesc, wc, wnt, pb, g2, be2,
              slab, buf=None):
    tiles = NSL // ROWS_TC
    off = slab * tiles
    grid = (tiles,)
    row_spec = lambda c: pl.BlockSpec((ROWS_TC, c), lambda i: (i, 0))
    full_spec = lambda a: pl.BlockSpec(a.shape, lambda i: (0, 0))
    in_specs = [
        row_spec(D), pl.BlockSpec((8, ROWS_TC), lambda i: (0, i + off)),
        full_spec(w1t), full_spec(onesr), full_spec(onesc),
        full_spec(wc), full_spec(wnt),
        full_spec(pb), full_spec(g2), full_spec(be2),
    ]
    args = [cat_emb, nft, w1t, onesr, onesc, wc, wnt, pb, g2, be2]
    kwargs = {}
    body = _tc_dense_body
    if buf is not None:
        in_specs.append(pl.BlockSpec(memory_space=pl.ANY))
        args.append(buf)
        kwargs["input_output_aliases"] = {10: 0}
        body = lambda *refs: _tc_dense_body(*refs[:10], refs[-1])
    return pl.pallas_call(
        body,
        grid=grid,
        in_specs=in_specs,
        out_specs=pl.BlockSpec((ROWS_TC, D), lambda i: (i + off, 0)),
        out_shape=jax.ShapeDtypeStruct((N, D), jnp.float32),
        **kwargs,
    )(*args)


def kernel(activities, resources, num_feats, act_table, res_table, num_w1,
           num_b1, ln1_g, ln1_b, proj_w, proj_b, ln2_g, ln2_b):
    ai = activities.astype(jnp.int32)
    ri = resources.astype(jnp.int32)
    cats = [_sc_gather_two(act_table, res_table,
                           ai[s * NSL:(s + 1) * NSL], ri[s * NSL:(s + 1) * NSL])
            for s in range(SLABS)]

    nft = jnp.concatenate(
        [num_feats.T, jnp.ones((1, N), jnp.float32),
         jnp.zeros((4, N), jnp.float32)], axis=0)
    w1t = jnp.concatenate(
        [num_w1, num_b1[None, :], jnp.zeros((4, D), jnp.float32)], axis=0).T

    mm = jnp.eye(D, dtype=jnp.float32) - 1.0 / D
    onesr = jnp.full((1, D), 1.0 / D, jnp.float32)
    onesc = jnp.full((D, 1), 1.0 / D, jnp.float32)

    wn_raw = proj_w[2 * H:]
    wc = proj_w[0:2 * H] @ mm
    wnt = ((ln1_g[:, None] * wn_raw) @ mm).T
    pb = ((ln1_b @ wn_raw + proj_b) @ mm)[None, :]
    g2 = ln2_g.reshape(1, D)
    b2 = ln2_b.reshape(1, D)
    out = None
    for s in range(SLABS):
        out = _tc_dense(cats[s], nft, w1t, onesr, onesc, wc, wnt, pb,
                        g2, b2, slab=s, buf=out)
    return out
